# baseline (speedup 1.0000x reference)
import functools

import jax
import jax.numpy as jnp
from jax import lax
from jax.experimental import pallas as pl
from jax.experimental.pallas import tpu as pltpu
from jax.experimental.pallas import tpu_sc as plsc

EMBED_DIM = 64
BATCH = 1024
SEQ_LEN = 200
N_TOK = BATCH * SEQ_LEN
N_WORKERS = 32
PER_W = N_TOK // N_WORKERS
CHUNK = 128
N_CHUNK = PER_W // CHUNK
N_GRP = CHUNK // 16
NBUF = 2
HALF = 128


def _sc_gather(table_hbm, pidx_hbm, idx_hbm, out_hbm, pidx_v, idx_v,
               rows0, rows1, ext0, ext1, gsem0, gsem1, osem0, osem1):
    wid = lax.axis_index("s") * 2 + lax.axis_index("c")
    obase = wid * (PER_W // 2)

    rows = [rows0, rows1]
    exts = [ext0, ext1]
    gsems = [gsem0, gsem1]
    osems = [osem0, osem1]

    ibase = pl.multiple_of(wid * PER_W, 8)
    pltpu.sync_copy(pidx_hbm.at[pl.ds(ibase, PER_W)], pidx_v)
    pltpu.sync_copy(idx_hbm.at[pl.ds(ibase, PER_W)], idx_v)

    iota = lax.broadcasted_iota(jnp.int32, (16,), 0)
    colb = (iota & 1) * EMBED_DIM

    def extract(j, b):
        def gbody(g, carry):
            j16 = g * 16
            ids = idx_v[pl.ds(j * CHUNK + j16, 16)]
            half = (ids & 1) * EMBED_DIM
            srow = iota + j16
            drow = srow >> 1
            for d in range(EMBED_DIM):
                vals = plsc.load_gather(rows[b], [srow, half + d])
                plsc.store_scatter(exts[b], [drow, colb + d], vals)
            return carry
        lax.fori_loop(0, N_GRP, gbody, 0)

    def cbody(i, carry):
        gcp = [None] * NBUF
        for b in range(NBUF):
            j = i * NBUF + b

            @pl.when(j >= NBUF)
            def _drain():
                pltpu.make_async_copy(
                    exts[b], out_hbm.at[pl.ds(0, CHUNK // 2)], osems[b]
                ).wait()

            joff = pl.multiple_of(j * CHUNK, 128)
            gcp[b] = pltpu.async_copy(
                table_hbm.at[pidx_v.at[pl.ds(joff, CHUNK)]], rows[b],
                gsems[b])
        for b in range(NBUF):
            j = i * NBUF + b
            gcp[b].wait()
            extract(j, b)
            off = pl.multiple_of(obase + j * (CHUNK // 2), 8)
            pltpu.async_copy(
                exts[b], out_hbm.at[pl.ds(off, CHUNK // 2)], osems[b])
        return carry

    lax.fori_loop(0, N_CHUNK // NBUF, cbody, 0)
    for b in range(NBUF):
        pltpu.make_async_copy(
            exts[b], out_hbm.at[pl.ds(0, CHUNK // 2)], osems[b]
        ).wait()


_gather_call = functools.partial(
    pl.kernel,
    mesh=plsc.VectorSubcoreMesh(core_axis_name="c", subcore_axis_name="s"),
    out_type=jax.ShapeDtypeStruct((N_TOK // 2, HALF), jnp.float32),
    scratch_types=[
        pltpu.VMEM((PER_W,), jnp.int32),
        pltpu.VMEM((PER_W,), jnp.int32),
        pltpu.VMEM((CHUNK, HALF), jnp.float32),
        pltpu.VMEM((CHUNK, HALF), jnp.float32),
        pltpu.VMEM((CHUNK // 2, HALF), jnp.float32),
        pltpu.VMEM((CHUNK // 2, HALF), jnp.float32),
        pltpu.SemaphoreType.DMA,
        pltpu.SemaphoreType.DMA,
        pltpu.SemaphoreType.DMA,
        pltpu.SemaphoreType.DMA,
    ],
    compiler_params=pltpu.CompilerParams(needs_layout_passes=False),
)(_sc_gather)


def _mask_body(tokens_ref, pad_ref, seq_ref):
    pad_ref[...] = tokens_ref[...] == 0
    row = lax.broadcasted_iota(jnp.int32, (SEQ_LEN, SEQ_LEN), 0)
    col = lax.broadcasted_iota(jnp.int32, (SEQ_LEN, SEQ_LEN), 1)
    seq_ref[...] = col > row


_mask_call = pl.pallas_call(
    _mask_body,
    out_shape=(
        jax.ShapeDtypeStruct((BATCH, SEQ_LEN), jnp.bool_),
        jax.ShapeDtypeStruct((SEQ_LEN, SEQ_LEN), jnp.bool_),
    ),
)


@jax.jit
def kernel(tokens, table):
    tokens = tokens.astype(jnp.int32)
    idx = tokens.reshape(N_TOK)
    table2 = table.reshape(500000, HALF)
    out2 = _gather_call(table2, idx >> 1, idx)
    features = out2.reshape(BATCH, SEQ_LEN, EMBED_DIM)
    pad, seq = _mask_call(tokens)
    return (features, (pad[:, None, :], seq))
